# baseline (speedup 1.0000x reference)
import jax
import jax.numpy as jnp
from jax import lax
from jax.experimental import pallas as pl
from jax.experimental.pallas import tpu as pltpu
from jax.experimental.pallas import tpu_sc as plsc

_N = 320000
_D = 128
_B = 1024
_STEPS = 3

_NC = 2
_NS = 16
_NW = _NC * _NS
_CHUNK = 128
_L = 16
_NJ = _D // _L
_T_TOTAL = 4096
_W_PER = _T_TOTAL // _NW
_P_PAD = 144


def _sc_body(x_hbm, q_hbm, basep, lop, hip, segp,
             mw_hbm, sw_hbm, vw_hbm,
             bp_v, lo_v, hi_v, sg_v, q8_0, q8_1, xb_0, xb_1,
             mw_acc, sw_acc, vw_acc, sem0, sem1):
    wid = lax.axis_index("s") * _NC + lax.axis_index("c")
    pltpu.sync_copy(basep.at[pl.ds(wid, 1)], bp_v)
    pltpu.sync_copy(lop.at[pl.ds(wid, 1)], lo_v)
    pltpu.sync_copy(hip.at[pl.ds(wid, 1)], hi_v)
    pltpu.sync_copy(segp.at[pl.ds(wid, 1)], sg_v)
    xbufs = (xb_0, xb_1)
    qbufs = (q8_0, q8_1)
    sems = (sem0, sem1)
    zero = jnp.zeros((_L,), jnp.float32)
    m_init = jnp.full((_L,), -3.0e38, jnp.float32)
    ninf_bits = jnp.full((_L,), -8388608, jnp.int32)
    iota = lax.iota(jnp.int32, _L)
    perms = [jnp.bitwise_xor(iota, k) for k in (8, 4, 2, 1)]

    dnums = lax.GatherDimensionNumbers(
        offset_dims=(), collapsed_slice_dims=(0,), start_index_map=(0,))

    def lane_sum(v):
        for p in perms:
            v = v + lax.gather(v, p[:, None], dnums, (1,),
                               mode=lax.GatherScatterMode.PROMISE_IN_BOUNDS)
        return v

    def params_at(t):
        base = pl.multiple_of(bp_v[0, pl.ds(t, _L)][0], _CHUNK)
        lo = lo_v[0, pl.ds(t, _L)][0]
        hi = hi_v[0, pl.ds(t, _L)][0]
        seg = jnp.minimum(sg_v[0, pl.ds(t, _L)][0], _B - 1)
        return base, lo, hi, seg

    def start_win(t, b):
        base, _, _, seg = params_at(t)
        qb = pl.multiple_of(seg - lax.rem(seg, 8), 8)
        pltpu.async_copy(x_hbm.at[pl.ds(base, _CHUNK)], xbufs[b], sems[b])
        pltpu.async_copy(q_hbm.at[pl.ds(qb, 8)], qbufs[b], sems[b])

    def wait_win(b):
        pltpu.make_async_copy(
            x_hbm.at[pl.ds(0, _CHUNK)], xbufs[b], sems[b]).wait()
        pltpu.make_async_copy(
            q_hbm.at[pl.ds(0, 8)], qbufs[b], sems[b]).wait()

    def proc_win(t, b):
        _, lo, hi, seg = params_at(t)
        qoff = lax.rem(seg, 8)
        xb = xbufs[b]
        qv = [qbufs[b][qoff, pl.ds(_L * j, _L)] for j in range(_NJ)]
        lo_b = jnp.full((_L,), lo, jnp.int32)
        hi_b = jnp.full((_L,), hi, jnp.int32)
        carry0 = tuple(zero for _ in range(_NJ)) + (m_init, zero)

        def row_body(r, rc):
            v = rc[:_NJ]
            m, s = rc[_NJ], rc[_NJ + 1]
            xr = [xb[r, pl.ds(_L * j, _L)] for j in range(_NJ)]
            t0 = [xr[j] * qv[j] for j in range(_NJ)]
            acc = (((t0[0] + t0[1]) + (t0[2] + t0[3]))
                   + ((t0[4] + t0[5]) + (t0[6] + t0[7])))
            e = lane_sum(acc)
            r_b = jnp.full((_L,), r, jnp.int32)
            neg = jnp.minimum(r_b - lo_b, hi_b - r_b - 1) >> 31
            ei = lax.bitcast_convert_type(e, jnp.int32)
            e = lax.bitcast_convert_type((ei & ~neg) | (ninf_bits & neg),
                                         jnp.float32)
            m_new = jnp.maximum(m, e)
            wo = jnp.exp(m - m_new)
            wr = jnp.exp(e - m_new)
            s_new = s * wo + wr
            v_new = tuple(v[j] * wo + wr * xr[j] for j in range(_NJ))
            return v_new + (m_new, s_new)

        rc = lax.fori_loop(0, _CHUNK, row_body, carry0, unroll=32)
        for j in range(_NJ):
            vw_acc[t, pl.ds(_L * j, _L)] = rc[j]
        mw_acc[t, :] = rc[_NJ]
        sw_acc[t, :] = rc[_NJ + 1]

    start_win(jnp.int32(0), 0)

    def pair_loop(k, carry_unused):
        t0 = 2 * k
        start_win(t0 + 1, 1)
        wait_win(0)
        proc_win(t0, 0)
        start_win(t0 + 2, 0)
        wait_win(1)
        proc_win(t0 + 1, 1)
        return carry_unused

    lax.fori_loop(0, _W_PER // 2, pair_loop, jnp.int32(0))
    wait_win(0)
    out0 = pl.multiple_of(wid * _W_PER, _W_PER)
    pltpu.sync_copy(mw_acc, mw_hbm.at[pl.ds(out0, _W_PER)])
    pltpu.sync_copy(sw_acc, sw_hbm.at[pl.ds(out0, _W_PER)])
    pltpu.sync_copy(vw_acc, vw_hbm.at[pl.ds(out0, _W_PER)])


def _sc_attention(x, q, basep, lop, hip, segp):
    kern = pl.kernel(
        _sc_body,
        out_type=(jax.ShapeDtypeStruct((_T_TOTAL, _L), jnp.float32),
                  jax.ShapeDtypeStruct((_T_TOTAL, _L), jnp.float32),
                  jax.ShapeDtypeStruct((_T_TOTAL, _D), jnp.float32)),
        mesh=plsc.VectorSubcoreMesh(core_axis_name="c", subcore_axis_name="s"),
        scratch_types=[
            pltpu.VMEM((1, _P_PAD), jnp.int32),
            pltpu.VMEM((1, _P_PAD), jnp.int32),
            pltpu.VMEM((1, _P_PAD), jnp.int32),
            pltpu.VMEM((1, _P_PAD), jnp.int32),
            pltpu.VMEM((8, _D), jnp.float32),
            pltpu.VMEM((8, _D), jnp.float32),
            pltpu.VMEM((_CHUNK, _D), jnp.float32),
            pltpu.VMEM((_CHUNK, _D), jnp.float32),
            pltpu.VMEM((_W_PER, _L), jnp.float32),
            pltpu.VMEM((_W_PER, _L), jnp.float32),
            pltpu.VMEM((_W_PER, _D), jnp.float32),
            pltpu.SemaphoreType.DMA,
            pltpu.SemaphoreType.DMA,
        ],
    )
    return kern(x, q, basep, lop, hip, segp)


def _merge_lstm_body(mw_ref, sw_ref, vw_ref, wseg_ref, h_ref, c_ref,
                     w1_ref, w2_ref, b_ref, q_out, c_out, r_out):
    segids = lax.broadcasted_iota(jnp.int32, (1, _B), 1)
    oh = (wseg_ref[:] == segids).astype(jnp.float32)
    mw = mw_ref[:]
    m_big = jnp.where(oh > 0.0, jnp.broadcast_to(mw, (_T_TOTAL, _B)), -3.0e38)
    m_seg = jnp.max(m_big, axis=0, keepdims=True)
    m_win = jnp.dot(oh, m_seg.reshape(_B, 1),
                    preferred_element_type=jnp.float32)
    scale = jnp.exp(mw - m_win)
    sv = jnp.concatenate([sw_ref[:] * scale, vw_ref[:] * scale], axis=1)
    merged = lax.dot_general(oh, sv, (((0,), (0,)), ((), ())),
                             preferred_element_type=jnp.float32)
    s_m = merged[:, :1]
    v_m = merged[:, 1:]
    s_safe = jnp.where(s_m > 0.0, s_m, 1.0)
    r = v_m * (1.0 / s_safe)
    r_out[:] = r
    gates = (jnp.dot(h_ref[:], w1_ref[:], preferred_element_type=jnp.float32)
             + jnp.dot(r, w2_ref[:], preferred_element_type=jnp.float32)
             + b_ref[:])
    gi = jax.nn.sigmoid(gates[:, :_D])
    gf = jax.nn.sigmoid(gates[:, _D:2 * _D])
    gg = jnp.tanh(gates[:, 2 * _D:3 * _D])
    go = jax.nn.sigmoid(gates[:, 3 * _D:])
    c_new = gf * c_ref[:] + gi * gg
    c_out[:] = c_new
    q_out[:] = go * jnp.tanh(c_new)


def _merge_lstm(mw, sw, vw, wseg, h, c, w1t, w2t, b2):
    return pl.pallas_call(
        _merge_lstm_body,
        out_shape=(jax.ShapeDtypeStruct((_B, _D), jnp.float32),
                   jax.ShapeDtypeStruct((_B, _D), jnp.float32),
                   jax.ShapeDtypeStruct((_B, _D), jnp.float32)),
    )(mw, sw, vw, wseg, h, c, w1t, w2t, b2)


def _build_worklist(batch):
    off = jnp.searchsorted(
        batch, jnp.arange(_B + 1, dtype=jnp.int32), side="left"
    ).astype(jnp.int32)
    lo_seg, hi_seg = off[:-1], off[1:]
    n_s = hi_seg - lo_seg
    first_cell = lo_seg // _CHUNK
    last_cell = (hi_seg - 1) // _CHUNK
    cellcount = jnp.where(n_s > 0, last_cell - first_cell + 1, 0)
    total = jnp.sum(cellcount)
    starts = jnp.cumsum(cellcount) - cellcount
    cellcount_ext = jnp.concatenate(
        [cellcount, (_T_TOTAL - total)[None]]).astype(jnp.int32)
    wseg = jnp.repeat(jnp.arange(_B + 1, dtype=jnp.int32), cellcount_ext,
                      total_repeat_length=_T_TOTAL)
    starts_ext = jnp.concatenate([starts, total[None]]).astype(jnp.int32)
    first_ext = jnp.concatenate([first_cell, jnp.zeros((1,), jnp.int32)])
    lo_ext = jnp.concatenate([lo_seg, jnp.zeros((1,), jnp.int32)])
    hi_ext = jnp.concatenate([hi_seg, jnp.zeros((1,), jnp.int32)])
    within = jnp.arange(_T_TOTAL, dtype=jnp.int32) - starts_ext[wseg]
    cell = first_ext[wseg] + within
    base = cell * _CHUNK
    lo = jnp.clip(lo_ext[wseg] - base, 0, _CHUNK)
    hi = jnp.clip(hi_ext[wseg] - base, 0, _CHUNK)

    def pack(a):
        a2 = a.reshape(_NW, _W_PER)
        return jnp.pad(a2, ((0, 0), (0, _P_PAD - _W_PER)))

    return pack(base), pack(lo), pack(hi), pack(wseg), wseg.reshape(_T_TOTAL, 1)


def kernel(x, batch, W_ih, W_hh, b_ih, b_hh):
    x = x.astype(jnp.float32)
    batch = batch.astype(jnp.int32)
    basep, lop, hip, segp, wseg2 = _build_worklist(batch)
    w1t = (W_ih[:, :_D] + W_hh).T
    w2t = W_ih[:, _D:].T
    b2 = (b_ih + b_hh).reshape(1, 4 * _D)
    h = jnp.zeros((_B, _D), jnp.float32)
    c = jnp.zeros((_B, _D), jnp.float32)
    mw = jnp.full((_T_TOTAL, 1), -3.0e38, jnp.float32)
    sw = jnp.zeros((_T_TOTAL, 1), jnp.float32)
    vw = jnp.zeros((_T_TOTAL, _D), jnp.float32)
    for _ in range(_STEPS):
        h, c, _ = _merge_lstm(mw, sw, vw, wseg2, h, c, w1t, w2t, b2)
        mw16, sw16, vw = _sc_attention(x, h, basep, lop, hip, segp)
        mw = mw16[:, :1]
        sw = sw16[:, :1]
    _, _, r = _merge_lstm(mw, sw, vw, wseg2, h, c, w1t, w2t, b2)
    return jnp.concatenate([h, r], axis=-1)


# speedup vs baseline: 1.1666x; 1.1666x over previous
import jax
import jax.numpy as jnp
from jax import lax
from jax.experimental import pallas as pl
from jax.experimental.pallas import tpu as pltpu
from jax.experimental.pallas import tpu_sc as plsc

_N = 320000
_D = 128
_B = 1024
_STEPS = 3

_NC = 2
_NS = 16
_NW = _NC * _NS
_CHUNK = 128
_L = 16
_NJ = _D // _L
_T_TOTAL = 4096
_W_PER = _T_TOTAL // _NW
_P_PAD = 144


def _sc_body(x_hbm, q_hbm, basep, lop, hip, segp,
             mw_hbm, sw_hbm, vw_hbm,
             bp_v, lo_v, hi_v, sg_v, q8_0, q8_1, xb_0, xb_1,
             mw_acc, sw_acc, vw_acc, sem0, sem1):
    wid = lax.axis_index("s") * _NC + lax.axis_index("c")
    pltpu.sync_copy(basep.at[pl.ds(wid, 1)], bp_v)
    pltpu.sync_copy(lop.at[pl.ds(wid, 1)], lo_v)
    pltpu.sync_copy(hip.at[pl.ds(wid, 1)], hi_v)
    pltpu.sync_copy(segp.at[pl.ds(wid, 1)], sg_v)
    xbufs = (xb_0, xb_1)
    qbufs = (q8_0, q8_1)
    sems = (sem0, sem1)
    zero = jnp.zeros((_L,), jnp.float32)
    m_init = jnp.full((_L,), -3.0e38, jnp.float32)
    ninf_bits = jnp.full((_L,), -8388608, jnp.int32)
    iota = lax.iota(jnp.int32, _L)
    perms = [jnp.bitwise_xor(iota, k) for k in (8, 4, 2, 1)]

    dnums = lax.GatherDimensionNumbers(
        offset_dims=(), collapsed_slice_dims=(0,), start_index_map=(0,))

    def lane_sum(v):
        for p in perms:
            v = v + lax.gather(v, p[:, None], dnums, (1,),
                               mode=lax.GatherScatterMode.PROMISE_IN_BOUNDS)
        return v

    def params_at(t):
        base = pl.multiple_of(bp_v[0, pl.ds(t, _L)][0], _CHUNK)
        lo = lo_v[0, pl.ds(t, _L)][0]
        hi = hi_v[0, pl.ds(t, _L)][0]
        seg = jnp.minimum(sg_v[0, pl.ds(t, _L)][0], _B - 1)
        return base, lo, hi, seg

    def start_win(t, b):
        base, _, _, seg = params_at(t)
        qb = pl.multiple_of(seg - lax.rem(seg, 8), 8)
        pltpu.async_copy(x_hbm.at[pl.ds(base, _CHUNK)], xbufs[b], sems[b])
        pltpu.async_copy(q_hbm.at[pl.ds(qb, 8)], qbufs[b], sems[b])

    def wait_win(b):
        pltpu.make_async_copy(
            x_hbm.at[pl.ds(0, _CHUNK)], xbufs[b], sems[b]).wait()
        pltpu.make_async_copy(
            q_hbm.at[pl.ds(0, 8)], qbufs[b], sems[b]).wait()

    def proc_win(t, b):
        _, lo, hi, seg = params_at(t)
        qoff = lax.rem(seg, 8)
        xb = xbufs[b]
        qv = [qbufs[b][qoff, pl.ds(_L * j, _L)] for j in range(_NJ)]
        lo_b = jnp.full((_L,), lo, jnp.int32)
        hi_b = jnp.full((_L,), hi, jnp.int32)
        carry0 = tuple(zero for _ in range(_NJ)) + (m_init, zero)

        def row_body(r, rc):
            v = rc[:_NJ]
            m, s = rc[_NJ], rc[_NJ + 1]
            xr = [xb[r, pl.ds(_L * j, _L)] for j in range(_NJ)]
            t0 = [xr[j] * qv[j] for j in range(_NJ)]
            acc = (((t0[0] + t0[1]) + (t0[2] + t0[3]))
                   + ((t0[4] + t0[5]) + (t0[6] + t0[7])))
            e = lane_sum(acc)
            r_b = jnp.full((_L,), r, jnp.int32)
            neg = jnp.minimum(r_b - lo_b, hi_b - r_b - 1) >> 31
            ei = lax.bitcast_convert_type(e, jnp.int32)
            e = lax.bitcast_convert_type((ei & ~neg) | (ninf_bits & neg),
                                         jnp.float32)
            m_new = jnp.maximum(m, e)
            wo = jnp.exp(m - m_new)
            wr = jnp.exp(e - m_new)
            s_new = s * wo + wr
            v_new = tuple(v[j] * wo + wr * xr[j] for j in range(_NJ))
            return v_new + (m_new, s_new)

        rc = lax.fori_loop(0, _CHUNK, row_body, carry0, unroll=16)
        for j in range(_NJ):
            vw_acc[t, pl.ds(_L * j, _L)] = rc[j]
        mw_acc[t, :] = rc[_NJ]
        sw_acc[t, :] = rc[_NJ + 1]

    start_win(jnp.int32(0), 0)

    def pair_loop(k, carry_unused):
        t0 = 2 * k
        start_win(t0 + 1, 1)
        wait_win(0)
        proc_win(t0, 0)
        start_win(t0 + 2, 0)
        wait_win(1)
        proc_win(t0 + 1, 1)
        return carry_unused

    lax.fori_loop(0, _W_PER // 2, pair_loop, jnp.int32(0))
    wait_win(0)
    out0 = pl.multiple_of(wid * _W_PER, _W_PER)
    pltpu.sync_copy(mw_acc, mw_hbm.at[pl.ds(out0, _W_PER)])
    pltpu.sync_copy(sw_acc, sw_hbm.at[pl.ds(out0, _W_PER)])
    pltpu.sync_copy(vw_acc, vw_hbm.at[pl.ds(out0, _W_PER)])


def _sc_attention(x, q, basep, lop, hip, segp):
    kern = pl.kernel(
        _sc_body,
        out_type=(jax.ShapeDtypeStruct((_T_TOTAL, _L), jnp.float32),
                  jax.ShapeDtypeStruct((_T_TOTAL, _L), jnp.float32),
                  jax.ShapeDtypeStruct((_T_TOTAL, _D), jnp.float32)),
        mesh=plsc.VectorSubcoreMesh(core_axis_name="c", subcore_axis_name="s"),
        scratch_types=[
            pltpu.VMEM((1, _P_PAD), jnp.int32),
            pltpu.VMEM((1, _P_PAD), jnp.int32),
            pltpu.VMEM((1, _P_PAD), jnp.int32),
            pltpu.VMEM((1, _P_PAD), jnp.int32),
            pltpu.VMEM((8, _D), jnp.float32),
            pltpu.VMEM((8, _D), jnp.float32),
            pltpu.VMEM((_CHUNK, _D), jnp.float32),
            pltpu.VMEM((_CHUNK, _D), jnp.float32),
            pltpu.VMEM((_W_PER, _L), jnp.float32),
            pltpu.VMEM((_W_PER, _L), jnp.float32),
            pltpu.VMEM((_W_PER, _D), jnp.float32),
            pltpu.SemaphoreType.DMA,
            pltpu.SemaphoreType.DMA,
        ],
    )
    return kern(x, q, basep, lop, hip, segp)


def _merge_lstm_body(mw_ref, sw_ref, vw_ref, wseg_ref, h_ref, c_ref,
                     w1_ref, w2_ref, b_ref, q_out, c_out, r_out):
    segids = lax.broadcasted_iota(jnp.int32, (1, _B), 1)
    oh = (wseg_ref[:] == segids).astype(jnp.float32)
    mw = mw_ref[:]
    m_big = jnp.where(oh > 0.0, jnp.broadcast_to(mw, (_T_TOTAL, _B)), -3.0e38)
    m_seg = jnp.max(m_big, axis=0, keepdims=True)
    m_win = jnp.dot(oh, m_seg.reshape(_B, 1),
                    preferred_element_type=jnp.float32)
    scale = jnp.exp(mw - m_win)
    sv = jnp.concatenate([sw_ref[:] * scale, vw_ref[:] * scale], axis=1)
    merged = lax.dot_general(oh, sv, (((0,), (0,)), ((), ())),
                             preferred_element_type=jnp.float32)
    s_m = merged[:, :1]
    v_m = merged[:, 1:]
    s_safe = jnp.where(s_m > 0.0, s_m, 1.0)
    r = v_m * (1.0 / s_safe)
    r_out[:] = r
    gates = (jnp.dot(h_ref[:], w1_ref[:], preferred_element_type=jnp.float32)
             + jnp.dot(r, w2_ref[:], preferred_element_type=jnp.float32)
             + b_ref[:])
    gi = jax.nn.sigmoid(gates[:, :_D])
    gf = jax.nn.sigmoid(gates[:, _D:2 * _D])
    gg = jnp.tanh(gates[:, 2 * _D:3 * _D])
    go = jax.nn.sigmoid(gates[:, 3 * _D:])
    c_new = gf * c_ref[:] + gi * gg
    c_out[:] = c_new
    q_out[:] = go * jnp.tanh(c_new)


def _merge_lstm(mw, sw, vw, wseg, h, c, w1t, w2t, b2):
    return pl.pallas_call(
        _merge_lstm_body,
        out_shape=(jax.ShapeDtypeStruct((_B, _D), jnp.float32),
                   jax.ShapeDtypeStruct((_B, _D), jnp.float32),
                   jax.ShapeDtypeStruct((_B, _D), jnp.float32)),
    )(mw, sw, vw, wseg, h, c, w1t, w2t, b2)


def _build_worklist(batch):
    off = jnp.searchsorted(
        batch, jnp.arange(_B + 1, dtype=jnp.int32), side="left"
    ).astype(jnp.int32)
    lo_seg, hi_seg = off[:-1], off[1:]
    n_s = hi_seg - lo_seg
    first_cell = lo_seg // _CHUNK
    last_cell = (hi_seg - 1) // _CHUNK
    cellcount = jnp.where(n_s > 0, last_cell - first_cell + 1, 0)
    total = jnp.sum(cellcount)
    starts = jnp.cumsum(cellcount) - cellcount
    cellcount_ext = jnp.concatenate(
        [cellcount, (_T_TOTAL - total)[None]]).astype(jnp.int32)
    wseg = jnp.repeat(jnp.arange(_B + 1, dtype=jnp.int32), cellcount_ext,
                      total_repeat_length=_T_TOTAL)
    starts_ext = jnp.concatenate([starts, total[None]]).astype(jnp.int32)
    first_ext = jnp.concatenate([first_cell, jnp.zeros((1,), jnp.int32)])
    lo_ext = jnp.concatenate([lo_seg, jnp.zeros((1,), jnp.int32)])
    hi_ext = jnp.concatenate([hi_seg, jnp.zeros((1,), jnp.int32)])
    within = jnp.arange(_T_TOTAL, dtype=jnp.int32) - starts_ext[wseg]
    cell = first_ext[wseg] + within
    base = cell * _CHUNK
    lo = jnp.clip(lo_ext[wseg] - base, 0, _CHUNK)
    hi = jnp.clip(hi_ext[wseg] - base, 0, _CHUNK)

    def pack(a):
        a2 = a.reshape(_NW, _W_PER)
        return jnp.pad(a2, ((0, 0), (0, _P_PAD - _W_PER)))

    return pack(base), pack(lo), pack(hi), pack(wseg), wseg.reshape(_T_TOTAL, 1)


def kernel(x, batch, W_ih, W_hh, b_ih, b_hh):
    x = x.astype(jnp.float32)
    batch = batch.astype(jnp.int32)
    basep, lop, hip, segp, wseg2 = _build_worklist(batch)
    w1t = (W_ih[:, :_D] + W_hh).T
    w2t = W_ih[:, _D:].T
    b2 = (b_ih + b_hh).reshape(1, 4 * _D)
    h = jnp.zeros((_B, _D), jnp.float32)
    c = jnp.zeros((_B, _D), jnp.float32)
    mw = jnp.full((_T_TOTAL, 1), -3.0e38, jnp.float32)
    sw = jnp.zeros((_T_TOTAL, 1), jnp.float32)
    vw = jnp.zeros((_T_TOTAL, _D), jnp.float32)
    for _ in range(_STEPS):
        h, c, _ = _merge_lstm(mw, sw, vw, wseg2, h, c, w1t, w2t, b2)
        mw16, sw16, vw = _sc_attention(x, h, basep, lop, hip, segp)
        mw = mw16[:, :1]
        sw = sw16[:, :1]
    _, _, r = _merge_lstm(mw, sw, vw, wseg2, h, c, w1t, w2t, b2)
    return jnp.concatenate([h, r], axis=-1)
